# baseline (speedup 1.0000x reference)
import functools

import numpy as np
import jax
import jax.numpy as jnp
from jax import lax
from jax.experimental import pallas as pl
from jax.experimental.pallas import tpu as pltpu
from jax.experimental.pallas import tpu_sc as plsc

B, S, D = 4, 4096, 2048
N = B * S

_INFO = plsc.get_sparse_core_info()
NC, NS = _INFO.num_cores, _INFO.num_subcores
NW = NC * NS
ROWS_PER_W = N // NW
KS = 16
KD = 8
CS = 16
CD = 2 * CS
SA = CS * KS
SD = CD * KD
assert SA + SD == ROWS_PER_W


def _global_indices() -> np.ndarray:
    keys = jax.random.split(jax.random.key(42), B)
    perm = jax.vmap(lambda k: jax.random.permutation(k, S))(keys)
    glob = perm.astype(jnp.int32) + (jnp.arange(B, dtype=jnp.int32)[:, None] * S)
    return np.asarray(jax.device_get(glob), dtype=np.int32).reshape(-1)


_GIDX = _global_indices()


@functools.partial(
    pl.kernel,
    mesh=plsc.VectorSubcoreMesh(core_axis_name="c", subcore_axis_name="s"),
    out_type=jax.ShapeDtypeStruct((N, D), jnp.float32),
    scratch_types=[
        pltpu.VMEM((ROWS_PER_W,), jnp.int32),
        pltpu.SMEM((ROWS_PER_W,), jnp.int32),
        pltpu.VMEM_SHARED((NS, ROWS_PER_W), jnp.int32),
        pltpu.VMEM((KS, D), jnp.float32),
        pltpu.VMEM((KS, D), jnp.float32),
        pltpu.VMEM_SHARED((NS, 2, KD, D), jnp.float32),
        pltpu.SemaphoreType.DMA,
        pltpu.SemaphoreType.DMA,
        pltpu.SemaphoreType.DMA,
        pltpu.SemaphoreType.DMA,
        pltpu.SemaphoreType.DMA,
        pltpu.SemaphoreType.DMA,
        pltpu.SemaphoreType.DMA,
        pltpu.SemaphoreType.DMA,
        pltpu.SemaphoreType.DMA,
    ],
)
def _gather_rows(x_hbm, idx_hbm, out_hbm,
                 idx_v, idx_sm, idx_spm, sb0, sb1, dbuf,
                 sem_i, sg0, sg1, st0, st1, dg0, dg1, dt0, dt1):
    cid = lax.axis_index("c")
    sid = lax.axis_index("s")
    wid = sid * NC + cid
    base = wid * ROWS_PER_W

    pltpu.async_copy(idx_hbm.at[pl.ds(base, ROWS_PER_W)], idx_v, sem_i).wait()
    pltpu.async_copy(idx_hbm.at[pl.ds(base, ROWS_PER_W)], idx_spm.at[sid], sem_i).wait()
    pltpu.async_copy(idx_spm.at[sid], idx_sm, sem_i).wait()

    sbufs = (sb0, sb1)
    sgs = (sg0, sg1)
    sts = (st0, st1)
    dgs = (dg0, dg1)
    dts = (dt0, dt1)

    def s_gather(c, s):
        pltpu.async_copy(x_hbm.at[idx_v.at[pl.ds(c * KS, KS)]], sbufs[s], sgs[s])

    def s_gwait(s):
        pltpu.make_async_copy(x_hbm.at[pl.ds(0, KS)], sbufs[s], sgs[s]).wait()

    def s_store(c, s):
        pltpu.async_copy(sbufs[s], out_hbm.at[pl.ds(base + c * KS, KS)], sts[s])

    def s_swait(s):
        pltpu.make_async_copy(sbufs[s], out_hbm.at[pl.ds(base, KS)], sts[s]).wait()

    def d_gather(c, s):
        def row(i, carry):
            r = idx_sm[SA + c * KD + i]
            pltpu.async_copy(
                x_hbm.at[pl.ds(r, 1)], dbuf.at[sid, s, pl.ds(i, 1)], dgs[s]
            )
            return carry

        lax.fori_loop(0, KD, row, 0)

    def d_gwait(s):
        pltpu.make_async_copy(x_hbm.at[pl.ds(0, KD)], dbuf.at[sid, s], dgs[s]).wait()

    def d_store(c, s):
        pltpu.async_copy(
            dbuf.at[sid, s], out_hbm.at[pl.ds(base + SA + c * KD, KD)], dts[s]
        )

    def d_swait(s):
        pltpu.make_async_copy(
            dbuf.at[sid, s], out_hbm.at[pl.ds(base, KD)], dts[s]
        ).wait()

    def hs_s(t, sl, first=False, last=False):
        s_gwait(sl)
        s_store(t, sl)
        if not first:
            s_swait(1 - sl)
        if not last:
            s_gather(t + 1, 1 - sl)

    def hs_d(u, sl, first=False, last=False):
        d_gwait(sl)
        d_store(u, sl)
        if not first:
            d_swait(1 - sl)
        if not last:
            d_gather(u + 1, 1 - sl)

    s_gather(0, 0)
    d_gather(0, 0)
    hs_s(0, 0, first=True); hs_d(0, 0, first=True); hs_d(1, 1)
    hs_s(1, 1);             hs_d(2, 0);             hs_d(3, 1)

    def pair(i, carry):
        t = i * 2
        u = i * 4
        hs_s(t, 0);     hs_d(u, 0);     hs_d(u + 1, 1)
        hs_s(t + 1, 1); hs_d(u + 2, 0); hs_d(u + 3, 1)
        return carry

    lax.fori_loop(1, CS // 2 - 1, pair, 0)

    t = CS - 2
    u = CD - 4
    hs_s(t, 0);                hs_d(u, 0);     hs_d(u + 1, 1)
    hs_s(t + 1, 1, last=True); hs_d(u + 2, 0); hs_d(u + 3, 1, last=True)
    s_swait(1)
    d_swait(1)


def kernel(X):
    gidx = jnp.asarray(_GIDX)
    out = _gather_rows(X.reshape(N, D), gidx)
    return out.reshape(B, S, D)


# speedup vs baseline: 1.0429x; 1.0429x over previous
import functools

import numpy as np
import jax
import jax.numpy as jnp
from jax import lax
from jax.experimental import pallas as pl
from jax.experimental.pallas import tpu as pltpu
from jax.experimental.pallas import tpu_sc as plsc

B, S, D = 4, 4096, 2048
N = B * S

_INFO = plsc.get_sparse_core_info()
NC, NS = _INFO.num_cores, _INFO.num_subcores
NW = NC * NS
ROWS_PER_W = N // NW
K = 16
N_CHUNKS = ROWS_PER_W // K


def _global_indices() -> np.ndarray:
    keys = jax.random.split(jax.random.key(42), B)
    perm = jax.vmap(lambda k: jax.random.permutation(k, S))(keys)
    glob = perm.astype(jnp.int32) + (jnp.arange(B, dtype=jnp.int32)[:, None] * S)
    return np.asarray(jax.device_get(glob), dtype=np.int32).reshape(-1)


_GIDX = _global_indices()


@functools.partial(
    pl.kernel,
    mesh=plsc.VectorSubcoreMesh(core_axis_name="c", subcore_axis_name="s"),
    out_type=jax.ShapeDtypeStruct((N, D), jnp.float32),
    scratch_types=[
        pltpu.SMEM((ROWS_PER_W,), jnp.int32),
        pltpu.VMEM_SHARED((NS, ROWS_PER_W), jnp.int32),
        pltpu.VMEM_SHARED((NS, 3, K, D), jnp.float32),
        pltpu.SemaphoreType.DMA,
        pltpu.SemaphoreType.DMA,
        pltpu.SemaphoreType.DMA,
        pltpu.SemaphoreType.DMA,
        pltpu.SemaphoreType.DMA,
        pltpu.SemaphoreType.DMA,
        pltpu.SemaphoreType.DMA,
    ],
)
def _gather_rows(x_hbm, idx_hbm, out_hbm, idx_sm, idx_spm, spm,
                 sem_i, dg0, dg1, dg2, do0, do1, do2):
    cid = lax.axis_index("c")
    sid = lax.axis_index("s")
    wid = sid * NC + cid
    base = wid * ROWS_PER_W
    pltpu.async_copy(idx_hbm.at[pl.ds(base, ROWS_PER_W)], idx_spm.at[sid], sem_i).wait()
    pltpu.async_copy(idx_spm.at[sid], idx_sm, sem_i).wait()

    gsems = (dg0, dg1, dg2)
    ssems = (do0, do1, do2)

    def start_gather(c, s):
        off = c * K
        for i in range(K):
            r = idx_sm[off + i]
            pltpu.async_copy(
                x_hbm.at[pl.ds(r, 1)], spm.at[sid, s, pl.ds(i, 1)], gsems[s]
            )

    def wait_gather(s):
        pltpu.make_async_copy(x_hbm.at[pl.ds(0, K)], spm.at[sid, s], gsems[s]).wait()

    def start_store(c, s):
        pltpu.async_copy(spm.at[sid, s], out_hbm.at[pl.ds(base + c * K, K)], ssems[s])

    def wait_store(s):
        pltpu.make_async_copy(spm.at[sid, s], out_hbm.at[pl.ds(base, K)], ssems[s]).wait()

    start_gather(0, 0)
    start_gather(1, 1)
    wait_gather(0)
    start_store(0, 0)
    start_gather(2, 2)
    wait_gather(1)
    start_store(1, 1)
    wait_store(0)
    start_gather(3, 0)

    def three(i, carry):
        c = 2 + i * 3
        for j, s in ((0, 2), (1, 0), (2, 1)):
            wait_gather(s)
            start_store(c + j, s)
            wait_store((s + 2) % 3)
            start_gather(c + j + 2, (s + 2) % 3)
        return carry

    lax.fori_loop(0, (N_CHUNKS - 5) // 3, three, 0)

    c = N_CHUNKS - 3
    s = c % 3
    wait_gather(s)
    start_store(c, s)
    wait_store((s + 2) % 3)
    start_gather(c + 2, (s + 2) % 3)
    wait_gather((s + 1) % 3)
    start_store(c + 1, (s + 1) % 3)
    wait_gather((s + 2) % 3)
    start_store(c + 2, (s + 2) % 3)
    wait_store(s)
    wait_store((s + 1) % 3)
    wait_store((s + 2) % 3)


def kernel(X):
    gidx = jnp.asarray(_GIDX)
    out = _gather_rows(X.reshape(N, D), gidx)
    return out.reshape(B, S, D)


# speedup vs baseline: 1.0583x; 1.0148x over previous
import functools

import numpy as np
import jax
import jax.numpy as jnp
from jax import lax
from jax.experimental import pallas as pl
from jax.experimental.pallas import tpu as pltpu
from jax.experimental.pallas import tpu_sc as plsc

B, S, D = 4, 4096, 2048
N = B * S

_INFO = plsc.get_sparse_core_info()
NC, NS = _INFO.num_cores, _INFO.num_subcores
NW = NC * NS
ROWS_PER_W = N // NW
K = 16
N_CHUNKS = ROWS_PER_W // K


def _global_indices() -> np.ndarray:
    keys = jax.random.split(jax.random.key(42), B)
    perm = jax.vmap(lambda k: jax.random.permutation(k, S))(keys)
    glob = perm.astype(jnp.int32) + (jnp.arange(B, dtype=jnp.int32)[:, None] * S)
    return np.asarray(jax.device_get(glob), dtype=np.int32).reshape(-1)


_GIDX = _global_indices()


@functools.partial(
    pl.kernel,
    mesh=plsc.VectorSubcoreMesh(core_axis_name="c", subcore_axis_name="s"),
    out_type=jax.ShapeDtypeStruct((N, D), jnp.float32),
    scratch_types=[
        pltpu.SMEM((ROWS_PER_W,), jnp.int32),
        pltpu.VMEM_SHARED((NS, ROWS_PER_W), jnp.int32),
        pltpu.VMEM_SHARED((NS, 3, K, D), jnp.float32),
        pltpu.SemaphoreType.DMA,
        pltpu.SemaphoreType.DMA,
        pltpu.SemaphoreType.DMA,
        pltpu.SemaphoreType.DMA,
        pltpu.SemaphoreType.DMA,
        pltpu.SemaphoreType.DMA,
        pltpu.SemaphoreType.DMA,
    ],
)
def _gather_rows(x_hbm, idx_hbm, out_hbm, idx_sm, idx_spm, spm,
                 sem_i, dg0, dg1, dg2, do0, do1, do2):
    cid = lax.axis_index("c")
    sid = lax.axis_index("s")
    wid = sid * NC + cid
    base = wid * ROWS_PER_W
    pltpu.async_copy(idx_hbm.at[pl.ds(base, ROWS_PER_W)], idx_spm.at[sid], sem_i).wait()
    pltpu.async_copy(idx_spm.at[sid], idx_sm, sem_i).wait()

    gsems = (dg0, dg1, dg2)
    ssems = (do0, do1, do2)

    def start_gather(c, s):
        def row(i, carry):
            r = idx_sm[c * K + i]
            pltpu.async_copy(
                x_hbm.at[pl.ds(r, 1)], spm.at[sid, s, pl.ds(i, 1)], gsems[s]
            )
            return carry

        lax.fori_loop(0, K, row, 0)

    def wait_gather(s):
        pltpu.make_async_copy(x_hbm.at[pl.ds(0, K)], spm.at[sid, s], gsems[s]).wait()

    def start_store(c, s):
        pltpu.async_copy(spm.at[sid, s], out_hbm.at[pl.ds(base + c * K, K)], ssems[s])

    def wait_store(s):
        pltpu.make_async_copy(spm.at[sid, s], out_hbm.at[pl.ds(base, K)], ssems[s]).wait()

    start_gather(0, 0)
    start_gather(1, 1)
    wait_gather(0)
    start_store(0, 0)
    start_gather(2, 2)
    wait_gather(1)
    start_store(1, 1)
    wait_store(0)
    start_gather(3, 0)

    def three(i, carry):
        c = 2 + i * 3
        for j, s in ((0, 2), (1, 0), (2, 1)):
            wait_store((s + 2) % 3)
            start_gather(c + j + 2, (s + 2) % 3)
            wait_gather(s)
            start_store(c + j, s)
        return carry

    lax.fori_loop(0, (N_CHUNKS - 5) // 3, three, 0)

    c = N_CHUNKS - 3
    s = c % 3
    wait_gather(s)
    start_store(c, s)
    wait_store((s + 2) % 3)
    start_gather(c + 2, (s + 2) % 3)
    wait_gather((s + 1) % 3)
    start_store(c + 1, (s + 1) % 3)
    wait_gather((s + 2) % 3)
    start_store(c + 2, (s + 2) % 3)
    wait_store(s)
    wait_store((s + 1) % 3)
    wait_store((s + 2) % 3)


def kernel(X):
    gidx = jnp.asarray(_GIDX)
    out = _gather_rows(X.reshape(N, D), gidx)
    return out.reshape(B, S, D)
